# baseline (speedup 1.0000x reference)
import functools

import jax
import jax.numpy as jnp
from jax import lax
from jax.experimental import pallas as pl
from jax.experimental.pallas import tpu as pltpu
from jax.experimental.pallas import tpu_sc as plsc

D_MODEL = 128
NUM_COLORS = 16
P = 4
PP = P * P
D_HALF = D_MODEL // 2

B, H, W = 8, 224, 224
PH, PW = H // P, W // P
NROWS = B * PH
NC, NS, L = 2, 16, 16
NW = NC * NS
ROWS_PER_W = NROWS // NW

X_ROW = P * W
O_ROW = PW * D_MODEL
T_LEN = PP * NUM_COLORS * D_HALF
NBUF = 4


def _round_bf16_bits(u):
    return (u + jnp.uint32(0x7FFF) + ((u >> 16) & jnp.uint32(1))) >> 16


def _table_body(emb_ref, w_ref, t_ref):
    e = emb_ref[...]
    for p in range(PP):
        w = w_ref[:, p * D_MODEL:(p + 1) * D_MODEL]
        t = lax.dot_general(e, w, (((1,), (1,)), ((), ())),
                            preferred_element_type=jnp.float32)
        lo = lax.bitcast_convert_type(t[:, :D_HALF], jnp.uint32)
        hi = lax.bitcast_convert_type(t[:, D_HALF:], jnp.uint32)
        word = _round_bf16_bits(lo) | (_round_bf16_bits(hi) << 16)
        t_ref[p * NUM_COLORS:(p + 1) * NUM_COLORS, :] = (
            lax.bitcast_convert_type(word, jnp.int32))


def _build_table(emb, w_proj):
    return pl.pallas_call(
        _table_body,
        out_shape=jax.ShapeDtypeStruct((PP * NUM_COLORS, D_HALF), jnp.int32),
    )(emb, w_proj)


@functools.cache
def _make_sc_embed():
    mesh = plsc.VectorSubcoreMesh(core_axis_name="c", subcore_axis_name="s")
    return functools.partial(
        pl.kernel,
        out_type=jax.ShapeDtypeStruct((NROWS * O_ROW,), jnp.float32),
        mesh=mesh,
        scratch_types=[
            pltpu.VMEM((T_LEN,), jnp.int32),
            pltpu.VMEM((ROWS_PER_W * X_ROW,), jnp.int32),
            pltpu.VMEM((NBUF * O_ROW,), jnp.float32),
            pltpu.SemaphoreType.DMA,
        ],
        compiler_params=pltpu.CompilerParams(needs_layout_passes=False),
    )(_sc_embed_body)


def _sc_embed_body(x_hbm, t_hbm, out_hbm, t_v, x_v, o_v, o_sem):
    wid = lax.axis_index("s") * NC + lax.axis_index("c")
    pltpu.sync_copy(t_hbm, t_v)
    pltpu.sync_copy(
        x_hbm.at[pl.ds(wid * (ROWS_PER_W * X_ROW), ROWS_PER_W * X_ROW)], x_v)
    lanes = lax.iota(jnp.int32, L)
    pos_x = (lanes // P) * W + (lanes % P)
    pos_t = lanes * (NUM_COLORS * D_HALF)
    splat_p = [jnp.full((L,), p, jnp.int32) for p in range(PP)]
    t_dg = [t_v.at[pl.ds(dg * L, T_LEN - dg * L)] for dg in range(D_HALF // L)]

    def o_copy(i, par):
        return pltpu.make_async_copy(
            o_v.at[pl.ds(par * O_ROW, O_ROW)],
            out_hbm.at[pl.ds((wid * ROWS_PER_W + i) * O_ROW, O_ROW)],
            o_sem)

    def row_body(i, _):
        par = lax.rem(i, NBUF)

        @pl.when(i >= NBUF)
        def _wait_reuse():
            o_copy(i - NBUF, par).wait()

        o_base = par * O_ROW
        x_base = i * X_ROW

        @plsc.parallel_loop(0, PW, unroll=8)
        def patch_body(pw):
            cv = plsc.load_gather(x_v, [pos_x + (pw * P + x_base)])
            tcode = cv * D_HALF + pos_t
            sp = [jnp.take_along_axis(tcode, splat_p[p], axis=0) + lanes
                  for p in range(PP)]
            for dg in range(D_HALF // L):
                ga, gb = [], []
                for p in range(PP):
                    g = plsc.load_gather(t_dg[dg], [sp[p]])
                    a, b_ = plsc.unpack(plsc.bitcast(g, jnp.bfloat16),
                                        format=plsc.PackFormat.INTERLEAVED)
                    ga.append(a)
                    gb.append(b_)
                for g in (ga, gb):
                    while len(g) > 1:
                        g[:] = [g[j] + g[j + 1] for j in range(0, len(g), 2)]
                base = o_base + pw * D_MODEL + dg * L
                o_v[pl.ds(base, L)] = ga[0]
                o_v[pl.ds(base + D_HALF, L)] = gb[0]

        o_copy(i, par).start()
        return _

    lax.fori_loop(0, ROWS_PER_W, row_body, None)
    for k in range(NBUF):
        i = ROWS_PER_W - NBUF + k
        o_copy(i, i % NBUF).wait()


def kernel(x, embed_table, W_proj):
    x2 = x.astype(jnp.int32).reshape(B * H * W)
    t = _build_table(embed_table, W_proj).reshape(T_LEN)
    out = _make_sc_embed()(x2, t)
    return out.reshape(B, PH, PW, D_MODEL)


# speedup vs baseline: 1.4917x; 1.4917x over previous
import functools

import jax
import jax.numpy as jnp
from jax import lax
from jax.experimental import pallas as pl
from jax.experimental.pallas import tpu as pltpu
from jax.experimental.pallas import tpu_sc as plsc

D_MODEL = 128
NUM_COLORS = 16
P = 4
PP = P * P
D_HALF = D_MODEL // 2

B, H, W = 8, 224, 224
PH, PW = H // P, W // P
NROWS = B * PH
NC, NS, L = 2, 16, 16
NW = NC * NS
ROWS_PER_W = NROWS // NW

X_ROW = P * W
O_ROW = PW * D_MODEL
T_LEN = PP * NUM_COLORS * D_HALF
NBUF = 4


def _round_bf16_bits(u):
    return (u + jnp.uint32(0x7FFF) + ((u >> 16) & jnp.uint32(1))) >> 16


def _table_body(emb_ref, w_ref, t_ref):
    e = emb_ref[...]
    for p in range(PP):
        w = w_ref[:, p * D_MODEL:(p + 1) * D_MODEL]
        t = lax.dot_general(e, w, (((1,), (1,)), ((), ())),
                            preferred_element_type=jnp.float32)
        lo = lax.bitcast_convert_type(t[:, :D_HALF], jnp.uint32)
        hi = lax.bitcast_convert_type(t[:, D_HALF:], jnp.uint32)
        word = _round_bf16_bits(lo) | (_round_bf16_bits(hi) << 16)
        t_ref[p * NUM_COLORS:(p + 1) * NUM_COLORS, :] = (
            lax.bitcast_convert_type(word, jnp.int32))


def _build_table(emb, w_proj):
    return pl.pallas_call(
        _table_body,
        out_shape=jax.ShapeDtypeStruct((PP * NUM_COLORS, D_HALF), jnp.int32),
    )(emb, w_proj)


@functools.cache
def _make_sc_embed():
    mesh = plsc.VectorSubcoreMesh(core_axis_name="c", subcore_axis_name="s")
    return functools.partial(
        pl.kernel,
        out_type=jax.ShapeDtypeStruct((NROWS * O_ROW,), jnp.float32),
        mesh=mesh,
        scratch_types=[
            pltpu.VMEM((T_LEN,), jnp.int32),
            pltpu.VMEM((ROWS_PER_W * X_ROW,), jnp.int32),
            pltpu.VMEM((NBUF * O_ROW,), jnp.float32),
            pltpu.SemaphoreType.DMA,
        ],
        compiler_params=pltpu.CompilerParams(needs_layout_passes=False),
    )(_sc_embed_body)


def _sc_embed_body(x_hbm, t_hbm, out_hbm, t_v, x_v, o_v, o_sem):
    wid = lax.axis_index("s") * NC + lax.axis_index("c")
    pltpu.sync_copy(t_hbm, t_v)
    pltpu.sync_copy(
        x_hbm.at[pl.ds(wid * (ROWS_PER_W * X_ROW), ROWS_PER_W * X_ROW)], x_v)
    lanes = lax.iota(jnp.int32, L)
    pos_x = (lanes // P) * W + (lanes % P)
    pos_t = lanes * (NUM_COLORS * D_HALF)
    splat_p = [jnp.full((L,), p, jnp.int32) for p in range(PP)]
    t_dg = [t_v.at[pl.ds(dg * L, T_LEN - dg * L)] for dg in range(D_HALF // L)]

    def o_copy(i, par):
        return pltpu.make_async_copy(
            o_v.at[pl.ds(par * O_ROW, O_ROW)],
            out_hbm.at[pl.ds((wid * ROWS_PER_W + i) * O_ROW, O_ROW)],
            o_sem)

    def row_body(i, _):
        par = lax.rem(i, NBUF)

        @pl.when(i >= NBUF)
        def _wait_reuse():
            o_copy(i - NBUF, par).wait()

        o_base = par * O_ROW
        x_base = i * X_ROW

        @plsc.parallel_loop(0, PW, unroll=4)
        def patch_body(pw):
            cv = plsc.load_gather(x_v, [pos_x + (pw * P + x_base)])
            tcode = cv * D_HALF + pos_t
            sp = [jnp.take_along_axis(tcode, splat_p[p], axis=0) + lanes
                  for p in range(PP)]
            for dg in range(D_HALF // L):
                g = [plsc.bitcast(plsc.load_gather(t_dg[dg], [sp[p]]),
                                  jnp.bfloat16) for p in range(PP)]
                while len(g) > 1:
                    g = [g[j] + g[j + 1] for j in range(0, len(g), 2)]
                a, b_ = plsc.unpack(g[0], format=plsc.PackFormat.INTERLEAVED)
                base = o_base + pw * D_MODEL + dg * L
                o_v[pl.ds(base, L)] = a
                o_v[pl.ds(base + D_HALF, L)] = b_

        o_copy(i, par).start()
        return _

    lax.fori_loop(0, ROWS_PER_W, row_body, None)
    for k in range(NBUF):
        i = ROWS_PER_W - NBUF + k
        o_copy(i, i % NBUF).wait()


def kernel(x, embed_table, W_proj):
    x2 = x.astype(jnp.int32).reshape(B * H * W)
    t = _build_table(embed_table, W_proj).reshape(T_LEN)
    out = _make_sc_embed()(x2, t)
    return out.reshape(B, PH, PW, D_MODEL)


# speedup vs baseline: 1.8216x; 1.2211x over previous
import functools

import jax
import jax.numpy as jnp
from jax import lax
from jax.experimental import pallas as pl
from jax.experimental.pallas import tpu as pltpu
from jax.experimental.pallas import tpu_sc as plsc

D_MODEL = 128
NUM_COLORS = 16
P = 4
PP = P * P
D_HALF = D_MODEL // 2

B, H, W = 8, 224, 224
PH, PW = H // P, W // P
NROWS = B * PH
NC, NS, L = 2, 16, 16
NW = NC * NS
ROWS_PER_W = NROWS // NW

X_ROW = P * W
O_ROW = PW * D_MODEL
T_LEN = PP * NUM_COLORS * D_HALF
NBUF = 4


def _round_bf16_bits(u):
    return (u + jnp.uint32(0x7FFF) + ((u >> 16) & jnp.uint32(1))) >> 16


def _table_body(emb_ref, w_ref, t_ref):
    e = emb_ref[...]
    for p in range(PP):
        w = w_ref[:, p * D_MODEL:(p + 1) * D_MODEL]
        t = lax.dot_general(e, w, (((1,), (1,)), ((), ())),
                            preferred_element_type=jnp.float32)
        lo = lax.bitcast_convert_type(t[:, :D_HALF], jnp.uint32)
        hi = lax.bitcast_convert_type(t[:, D_HALF:], jnp.uint32)
        word = _round_bf16_bits(lo) | (_round_bf16_bits(hi) << 16)
        t_ref[p * NUM_COLORS:(p + 1) * NUM_COLORS, :] = (
            lax.bitcast_convert_type(word, jnp.int32))


def _build_table(emb, w_proj):
    return pl.pallas_call(
        _table_body,
        out_shape=jax.ShapeDtypeStruct((PP * NUM_COLORS, D_HALF), jnp.int32),
    )(emb, w_proj)


@functools.cache
def _make_sc_embed():
    mesh = plsc.VectorSubcoreMesh(core_axis_name="c", subcore_axis_name="s")
    return functools.partial(
        pl.kernel,
        out_type=jax.ShapeDtypeStruct((NROWS * O_ROW,), jnp.float32),
        mesh=mesh,
        scratch_types=[
            pltpu.VMEM((T_LEN,), jnp.int32),
            pltpu.VMEM((ROWS_PER_W * X_ROW,), jnp.int32),
            pltpu.VMEM((NBUF * O_ROW,), jnp.float32),
            pltpu.SemaphoreType.DMA,
        ],
        compiler_params=pltpu.CompilerParams(needs_layout_passes=False),
    )(_sc_embed_body)


def _sc_embed_body(x_hbm, t_hbm, out_hbm, t_v, x_v, o_v, o_sem):
    wid = lax.axis_index("s") * NC + lax.axis_index("c")
    pltpu.sync_copy(t_hbm, t_v)
    pltpu.sync_copy(
        x_hbm.at[pl.ds(wid * (ROWS_PER_W * X_ROW), ROWS_PER_W * X_ROW)], x_v)
    lanes = lax.iota(jnp.int32, L)
    pos_x = (lanes // P) * W + (lanes % P)
    pos_t = lanes * (NUM_COLORS * D_HALF)
    splat_p = [jnp.full((L,), p, jnp.int32) for p in range(PP)]
    t_dg = [t_v.at[pl.ds(dg * L, T_LEN - dg * L)] for dg in range(D_HALF // L)]

    def o_copy(i, par):
        return pltpu.make_async_copy(
            o_v.at[pl.ds(par * O_ROW, O_ROW)],
            out_hbm.at[pl.ds((wid * ROWS_PER_W + i) * O_ROW, O_ROW)],
            o_sem)

    def row_body(i, _):
        par = lax.rem(i, NBUF)

        @pl.when(i >= NBUF)
        def _wait_reuse():
            o_copy(i - NBUF, par).wait()

        o_base = par * O_ROW
        x_base = i * X_ROW

        @plsc.parallel_loop(0, PW, unroll=2)
        def patch_body(pw):
            cv = plsc.load_gather(x_v, [pos_x + (pw * P + x_base)])
            tcode = cv * D_HALF + pos_t
            sp = [jnp.take_along_axis(tcode, splat_p[p], axis=0) + lanes
                  for p in range(PP)]
            for dg in range(D_HALF // L):
                g = [plsc.bitcast(plsc.load_gather(t_dg[dg], [sp[p]]),
                                  jnp.bfloat16) for p in range(PP)]
                while len(g) > 1:
                    g = [g[j] + g[j + 1] for j in range(0, len(g), 2)]
                a, b_ = plsc.unpack(g[0], format=plsc.PackFormat.INTERLEAVED)
                base = o_base + pw * D_MODEL + dg * L
                o_v[pl.ds(base, L)] = a
                o_v[pl.ds(base + D_HALF, L)] = b_

        o_copy(i, par).start()
        return _

    lax.fori_loop(0, ROWS_PER_W, row_body, None)
    for k in range(NBUF):
        i = ROWS_PER_W - NBUF + k
        o_copy(i, i % NBUF).wait()


def kernel(x, embed_table, W_proj):
    x2 = x.astype(jnp.int32).reshape(B * H * W)
    t = _build_table(embed_table, W_proj).reshape(T_LEN)
    out = _make_sc_embed()(x2, t)
    return out.reshape(B, PH, PW, D_MODEL)
